# baseline (speedup 1.0000x reference)
import functools

import jax
import jax.numpy as jnp
from jax import lax
from jax.experimental import pallas as pl
from jax.experimental.pallas import tpu as pltpu
from jax.experimental.pallas import tpu_sc as plsc

BATCH = 4
SEQ = 8192
N = BATCH * SEQ
D = 1024
NW = 32
ROWS_PER_W = N // NW
UNROLL = 16
NBLK = ROWS_PER_W // UNROLL


def _make_kernel():
    mesh = plsc.VectorSubcoreMesh(core_axis_name="c", subcore_axis_name="s")

    @functools.partial(
        pl.kernel,
        mesh=mesh,
        out_type=jax.ShapeDtypeStruct((N * D,), jnp.float32),
        scratch_types=[
            pltpu.VMEM((ROWS_PER_W,), jnp.int32),
            pltpu.VMEM((2 * D,), jnp.float32),
            pltpu.SemaphoreType.DMA,
        ],
    )
    def k(ids_hbm, table_hbm, out_hbm, idx_v, tab_v, sem):
        wid = lax.axis_index("s") * 2 + lax.axis_index("c")
        base = wid * ROWS_PER_W
        pltpu.sync_copy(ids_hbm.at[pl.ds(base, ROWS_PER_W)], idx_v)
        pltpu.sync_copy(table_hbm, tab_v)

        def body(blk, _):
            r0 = blk * UNROLL
            v = idx_v[pl.ds(r0, 16)]
            for j in range(UNROLL):
                src = tab_v.at[pl.ds(v[j] * D, D)]
                dst = out_hbm.at[pl.ds((base + r0 + j) * D, D)]
                pltpu.async_copy(src, dst, sem)
            return _

        lax.fori_loop(0, NBLK, body, None)
        pltpu.make_async_copy(
            out_hbm.at[pl.ds(base * D, ROWS_PER_W * D)],
            out_hbm.at[pl.ds(base * D, ROWS_PER_W * D)],
            sem,
        ).wait()

    return k


_k = _make_kernel()


def kernel(token_type_ids, table):
    ids_flat = token_type_ids.reshape(-1).astype(jnp.int32)
    out = _k(ids_flat, table.reshape(-1).astype(jnp.float32))
    return out.reshape(BATCH, SEQ, D)
